# Initial kernel scaffold; baseline (speedup 1.0000x reference)
import jax
import jax.numpy as jnp
from jax.experimental import pallas as pl


def kernel(x, edge_index, eps1, W1, b1, W2, b2, g1, be1, eps2, W3, b3, g2, be2, W4, b4):
    raise NotImplementedError("write your pallas kernel here")


# speedup vs baseline: 2.9246x; 2.9246x over previous
import functools
import math

import jax
import jax.numpy as jnp
from jax import lax
from jax.experimental import pallas as pl
from jax.experimental.pallas import tpu as pltpu
from jax.experimental.pallas import tpu_sc as plsc

N = 10000
E = 320000
D = 128
H = 512

NC = 2
NS = 16
LANES = 16

EDGES_PER_TILE = E // (NC * NS)
KB = 80
NBATCH = EDGES_PER_TILE // KB
NP = 10240
ROWS_PER_TILE = NP // NS
ZROWS = 128
BN_SCALE = 1.0 / math.sqrt(1.0 + 1e-5)


def _make_segsum(C):
    mesh = plsc.VectorSubcoreMesh(
        core_axis_name="c", subcore_axis_name="s", num_cores=NC, num_subcores=NS
    )

    def body(tab_hbm, src_hbm, dst_hbm, out_hbm, table, zero_v, srcb, dstb, rows, sem):
        cid = lax.axis_index("c")
        sid = lax.axis_index("s")
        ebase = (cid * NS + sid) * EDGES_PER_TILE

        zvec = jnp.zeros((LANES,), jnp.float32)

        def zfill(i, carry):
            r = i // (128 // LANES)
            j = i % (128 // LANES)
            zero_v[r, pl.ds(j * LANES, LANES)] = zvec
            return carry

        lax.fori_loop(0, ZROWS * (128 // LANES), zfill, 0)

        for c in range(C):
            def zcopy(j, carry):
                pltpu.sync_copy(
                    zero_v, table.at[pl.ds(sid * ROWS_PER_TILE + j * ZROWS, ZROWS)]
                )
                return carry

            lax.fori_loop(0, ROWS_PER_TILE // ZROWS, zcopy, 0)
            plsc.subcore_barrier()

            def ebody(b, carry):
                base = ebase + b * KB
                pltpu.sync_copy(src_hbm.at[pl.ds(base, KB)], srcb)
                pltpu.sync_copy(dst_hbm.at[pl.ds(base, KB)], dstb)
                if c > 0:
                    for i in range(KB // LANES):
                        sl = pl.ds(i * LANES, LANES)
                        srcb[sl] = srcb[sl] + c * N
                pltpu.async_copy(tab_hbm.at[srcb], rows, sem).wait()
                pltpu.sync_copy(rows, table.at[dstb], add=True)
                return carry

            lax.fori_loop(0, NBATCH, ebody, 0)
            plsc.subcore_barrier()

            pltpu.sync_copy(
                table.at[pl.ds(sid * ROWS_PER_TILE, ROWS_PER_TILE)],
                out_hbm.at[cid, c, pl.ds(sid * ROWS_PER_TILE, ROWS_PER_TILE)],
            )
            plsc.subcore_barrier()

    return functools.partial(
        pl.kernel,
        out_type=jax.ShapeDtypeStruct((NC, C, NP, 128), jnp.float32),
        mesh=mesh,
        scratch_types=[
            pltpu.VMEM_SHARED((NP, 128), jnp.float32),
            pltpu.VMEM((ZROWS, 128), jnp.float32),
            pltpu.VMEM((KB,), jnp.int32),
            pltpu.VMEM((KB,), jnp.int32),
            pltpu.VMEM((KB, 128), jnp.float32),
            pltpu.SemaphoreType.DMA,
        ],
    )(body)


_segsum1 = _make_segsum(1)
_segsum4 = _make_segsum(4)

BM = 1000


def _mlp1_body(eps_ref, x_ref, p_ref, w1_ref, b1_ref, w2_ref, b2_ref, g1_ref, be1_ref, out_ref):
    a = x_ref[...] * (1.0 + eps_ref[0, 0]) + p_ref[0] + p_ref[1]
    t = jnp.dot(a, w1_ref[...], preferred_element_type=jnp.float32)
    t = jnp.maximum(t + b1_ref[...], 0.0)
    t = jnp.dot(t, w2_ref[...], preferred_element_type=jnp.float32)
    t = jnp.maximum(t + b2_ref[...], 0.0)
    t = t * (g1_ref[...] * BN_SCALE) + be1_ref[...]
    for c in range(4):
        out_ref[c] = t[:, c * 128:(c + 1) * 128]


def _mlp2_body(eps_ref, h4_ref, p_ref, w3_ref, b3_ref, g2_ref, be2_ref, w4_ref, b4_ref, out_ref):
    t = None
    for c in range(4):
        hc = h4_ref[c] * (1.0 + eps_ref[0, 0]) + p_ref[0, c] + p_ref[1, c]
        tc = jnp.dot(hc, w3_ref[c * 128:(c + 1) * 128, :],
                     preferred_element_type=jnp.float32)
        t = tc if t is None else t + tc
    t = jnp.maximum(t + b3_ref[...], 0.0)
    t = t * (g2_ref[...] * BN_SCALE) + be2_ref[...]
    t = jnp.dot(t, w4_ref[...], preferred_element_type=jnp.float32)
    out_ref[...] = jnp.maximum(t + b4_ref[...], 0.0)


def _row(v):
    return v.reshape(1, -1)


def kernel(x, edge_index, eps1, W1, b1, W2, b2, g1, be1, eps2, W3, b3, g2, be2, W4, b4):
    src = edge_index[0].astype(jnp.int32)
    dst = edge_index[1].astype(jnp.int32)

    p1 = _segsum1(x, src, dst)
    p1 = p1[:, 0, :N, :]

    grid = (N // BM,)
    h4 = pl.pallas_call(
        _mlp1_body,
        grid=grid,
        in_specs=[
            pl.BlockSpec(memory_space=pltpu.SMEM),
            pl.BlockSpec((BM, D), lambda i: (i, 0)),
            pl.BlockSpec((NC, BM, D), lambda i: (0, i, 0)),
            pl.BlockSpec((D, H), lambda i: (0, 0)),
            pl.BlockSpec((1, H), lambda i: (0, 0)),
            pl.BlockSpec((H, H), lambda i: (0, 0)),
            pl.BlockSpec((1, H), lambda i: (0, 0)),
            pl.BlockSpec((1, H), lambda i: (0, 0)),
            pl.BlockSpec((1, H), lambda i: (0, 0)),
        ],
        out_specs=pl.BlockSpec((4, BM, D), lambda i: (0, i, 0)),
        out_shape=jax.ShapeDtypeStruct((4, N, D), jnp.float32),
    )(eps1.reshape(1, 1), x, p1, W1, _row(b1), W2, _row(b2), _row(g1), _row(be1))

    p2 = _segsum4(h4.reshape(4 * N, 128), src, dst)[:, :, :N, :]

    out = pl.pallas_call(
        _mlp2_body,
        grid=grid,
        in_specs=[
            pl.BlockSpec(memory_space=pltpu.SMEM),
            pl.BlockSpec((4, BM, D), lambda i: (0, i, 0)),
            pl.BlockSpec((NC, 4, BM, D), lambda i: (0, 0, i, 0)),
            pl.BlockSpec((H, H), lambda i: (0, 0)),
            pl.BlockSpec((1, H), lambda i: (0, 0)),
            pl.BlockSpec((1, H), lambda i: (0, 0)),
            pl.BlockSpec((1, H), lambda i: (0, 0)),
            pl.BlockSpec((H, H), lambda i: (0, 0)),
            pl.BlockSpec((1, H), lambda i: (0, 0)),
        ],
        out_specs=pl.BlockSpec((BM, H), lambda i: (i, 0)),
        out_shape=jax.ShapeDtypeStruct((N, H), jnp.float32),
    )(eps2.reshape(1, 1), h4, p2, W3, _row(b3), _row(g2), _row(be2), W4, _row(b4))

    return out
